# baseline (speedup 1.0000x reference)
import jax
import jax.numpy as jnp
from jax import lax
from jax.experimental import pallas as pl
from jax.experimental.pallas import tpu as pltpu

K_T = 3
K_S = 2
HIDDEN = 64
FK_DIM = 256
SEQ_LENGTH = 12
GAP = 2

_F32 = jnp.float32
_BF16 = jnp.bfloat16


def _dot(a, b):
    return jnp.dot(a, b, preferred_element_type=_F32)


def _dif_block_kernel(x_ref, xspa_ref, dyn_ref, wfcT_ref, wg_ref, wgbc_ref,
                      wgfk_ref, bg_ref, bbc2_ref, bfk2_ref, lns_ref, lnb_ref,
                      u_ref, fh_ref, xk_scr, dynp_scr, wins_scr, h9_scr):
    L = xspa_ref.shape[1]
    N = xspa_ref.shape[2]
    D = HIDDEN
    Lp = L - K_T + 1
    STEPS = SEQ_LENGTH // GAP - 1

    WfcT = wfcT_ref[...]
    Wg = wg_ref[...]
    Wgbc = wgbc_ref[...]
    WgFk = wgfk_ref[...]
    bg = bg_ref[...]
    bbc2 = bbc2_ref[...]
    bfk2 = bfk2_ref[...]
    lns = lns_ref[...]
    lnb = lnb_ref[...]

    dynb = dyn_ref[...].reshape(K_S * N, K_T * N).astype(_BF16)

    xsb = xspa_ref[0].astype(_BF16)
    xr = jnp.concatenate(
        [xsb[0:Lp], xsb[1:Lp + 1], xsb[2:Lp + 2]], axis=-1)
    tb = jnp.tanh(_dot(xr.reshape(Lp * N, K_T * D), WfcT)).astype(_BF16)
    x0 = ((tb[:, 0:D] + tb[:, D:2 * D] + tb[:, 2 * D:3 * D])
          * jnp.bfloat16(1.0 / K_T)).reshape(Lp, N, D)
    tb3 = tb.reshape(Lp, N, K_T * D)
    for l in range(Lp):
        for j in range(K_T):
            xk_scr[j * N:(j + 1) * N, l * D:(l + 1) * D] = tb3[l, :, j * D:(j + 1) * D]

    acb = _dot(dynb, xk_scr[...]).astype(_BF16)

    h9 = None
    for l in range(Lp):
        h = jnp.concatenate(
            [x0[l], acb[0:N, l * D:(l + 1) * D],
             acb[N:2 * N, l * D:(l + 1) * D]], axis=-1)
        if l == Lp - 1:
            h9 = h
        bc = _dot(h, Wgbc) + bbc2
        v = x_ref[0, l + K_T - 1] - jnp.maximum(bc, 0.0)
        mu = jnp.mean(v, axis=-1, keepdims=True)
        var = jnp.mean((v - mu) * (v - mu), axis=-1, keepdims=True)
        u_ref[0, l] = (v - mu) * jax.lax.rsqrt(var + 1e-5) * lns + lnb
    z9 = (_dot(h9, Wg) + bg).astype(_BF16)

    dynp = dynp_scr[...]
    fh_ref[0, 0] = _dot(h9_scr[...], WgFk) + bfk2

    def conv_one(w0, w1, w2):
        xr1 = jnp.concatenate([w0, w1, w2], axis=-1)
        t1 = jnp.tanh(_dot(xr1, WfcT)).astype(_BF16)
        x01 = (t1[:, 0:D] + t1[:, D:2 * D] + t1[:, 2 * D:3 * D]) * jnp.bfloat16(1.0 / K_T)
        xk1 = jnp.concatenate(
            [t1[:, j * D:(j + 1) * D] for j in range(K_T)], axis=0)
        ac = _dot(dynp, xk1).astype(_BF16)
        return jnp.concatenate([x01, ac[0:N], ac[N:2 * N]], axis=-1)

    wins = [wins_scr[0], wins_scr[1], wins_scr[2]]
    for s in range(STEPS):
        h1 = conv_one(wins[-3], wins[-2], wins[-1])
        f = (_dot(h1, Wg) + bg).astype(_BF16)
        fh_ref[0, s + 1] = _dot(h1, WgFk) + bfk2
        wins.append(f)

    dynp_scr[...] = dynb
    wins_scr[0] = xsb[L - 2]
    wins_scr[1] = xsb[L - 1]
    wins_scr[2] = z9
    h9_scr[...] = h9


def kernel(X, X_spa, dynamic_graph, static_graph, W_fc, W_gcn, b_gcn, W_bc,
           b_bc, W_fk, b_fk, ln_scale, ln_bias):
    B, L, N, D = X_spa.shape
    Lp = L - K_T + 1
    S = SEQ_LENGTH // GAP

    WgT = W_gcn.T
    args = (
        X,
        X_spa,
        dynamic_graph,
        W_fc.T.astype(_BF16),
        WgT.astype(_BF16),
        (WgT @ W_bc.T).astype(_BF16),
        (WgT @ W_fk.T).astype(_BF16),
        b_gcn.reshape(1, D),
        (b_gcn @ W_bc.T + b_bc).reshape(1, D),
        (b_gcn @ W_fk.T + b_fk).reshape(1, FK_DIM),
        ln_scale.reshape(1, D),
        ln_bias.reshape(1, D),
    )

    full = lambda shape: pl.BlockSpec(shape, lambda b: (0,) * len(shape))
    cur = lambda shape: pl.BlockSpec(
        (1,) + shape[1:], lambda b: (lax.rem(b, B),) + (0,) * (len(shape) - 1))

    in_specs = [
        cur(X.shape),
        cur(X_spa.shape),
        pl.BlockSpec((K_S, 1, N, K_T * N), lambda b: (0, lax.rem(b, B), 0, 0)),
        full((K_T * D, K_T * D)),
        full((K_T * D, D)),
        full((K_T * D, D)),
        full((K_T * D, FK_DIM)),
        full((1, D)),
        full((1, D)),
        full((1, FK_DIM)),
        full((1, D)),
        full((1, D)),
    ]
    out_specs = [
        cur((B, Lp, N, D)),
        pl.BlockSpec((1, S, N, FK_DIM),
                     lambda b: (lax.rem(b + B - 1, B), 0, 0, 0)),
    ]
    out_shape = [
        jax.ShapeDtypeStruct((B, Lp, N, D), _F32),
        jax.ShapeDtypeStruct((B, S, N, FK_DIM), _F32),
    ]

    u, fh = pl.pallas_call(
        _dif_block_kernel,
        grid=(B + 1,),
        in_specs=in_specs,
        out_specs=out_specs,
        out_shape=out_shape,
        scratch_shapes=[
            pltpu.VMEM((K_T * N, Lp * D), _BF16),
            pltpu.VMEM((K_S * N, K_T * N), _BF16),
            pltpu.VMEM((K_T, N, D), _BF16),
            pltpu.VMEM((N, K_T * D), _BF16),
        ],
        compiler_params=pltpu.CompilerParams(
            dimension_semantics=("arbitrary",),
        ),
    )(*args)
    return (u, fh)


# speedup vs baseline: 1.0405x; 1.0405x over previous
import jax
import jax.numpy as jnp
from jax.experimental import pallas as pl
from jax.experimental.pallas import tpu as pltpu

K_T = 3
K_S = 2
HIDDEN = 64
FK_DIM = 256
SEQ_LENGTH = 12
GAP = 2

_F32 = jnp.float32
_BF16 = jnp.bfloat16


def _dot(a, b):
    return jnp.dot(a, b, preferred_element_type=_F32)


def _dif_block_kernel(x_ref, xspa_ref, dyn_ref, wfcT_ref, wg_ref, wgbc_ref,
                      wgfk_ref, bg_ref, bbc2_ref, bfk2_ref, lns_ref, lnb_ref,
                      u_ref, fh_ref, xk_scr):
    L = xspa_ref.shape[1]
    N = xspa_ref.shape[2]
    D = HIDDEN
    Lp = L - K_T + 1
    STEPS = SEQ_LENGTH // GAP - 1

    WfcT = wfcT_ref[...]
    Wg = wg_ref[...]
    Wgbc = wgbc_ref[...]
    WgFk = wgfk_ref[...]
    bg = bg_ref[...]
    bbc2 = bbc2_ref[...]
    bfk2 = bfk2_ref[...]
    lns = lns_ref[...]
    lnb = lnb_ref[...]

    dynb = dyn_ref[...].reshape(K_S * N, K_T * N).astype(_BF16)

    xsb = xspa_ref[0]
    xr = jnp.concatenate(
        [xsb[0:Lp], xsb[1:Lp + 1], xsb[2:Lp + 2]], axis=-1)
    tb = jnp.tanh(_dot(xr.reshape(Lp * N, K_T * D), WfcT)).astype(_BF16)
    x0 = ((tb[:, 0:D] + tb[:, D:2 * D] + tb[:, 2 * D:3 * D])
          * jnp.bfloat16(1.0 / K_T)).reshape(Lp, N, D)
    tb3 = tb.reshape(Lp, N, K_T * D)
    for l in range(Lp):
        for j in range(K_T):
            xk_scr[j * N:(j + 1) * N, l * D:(l + 1) * D] = tb3[l, :, j * D:(j + 1) * D]

    acb = _dot(dynb, xk_scr[...]).astype(_BF16)

    def backcast_ln(l, h):
        bc = _dot(h, Wgbc) + bbc2
        v = x_ref[0, l + K_T - 1] - jnp.maximum(bc, 0.0)
        mu = jnp.mean(v, axis=-1, keepdims=True)
        var = jnp.mean((v - mu) * (v - mu), axis=-1, keepdims=True)
        u_ref[0, l] = (v - mu) * jax.lax.rsqrt(var + 1e-5) * lns + lnb

    def h_of_l(l):
        return jnp.concatenate(
            [x0[l], acb[0:N, l * D:(l + 1) * D],
             acb[N:2 * N, l * D:(l + 1) * D]], axis=-1)

    h9 = h_of_l(Lp - 1)
    z9 = (_dot(h9, Wg) + bg).astype(_BF16)
    fh_ref[0, 0] = _dot(h9, WgFk) + bfk2
    backcast_ln(Lp - 1, h9)

    def conv_one(w0, w1, w2):
        xr1 = jnp.concatenate([w0, w1, w2], axis=-1)
        t1 = jnp.tanh(_dot(xr1, WfcT)).astype(_BF16)
        x01 = (t1[:, 0:D] + t1[:, D:2 * D] + t1[:, 2 * D:3 * D]) * jnp.bfloat16(1.0 / K_T)
        xk1 = jnp.concatenate(
            [t1[:, j * D:(j + 1) * D] for j in range(K_T)], axis=0)
        ac = _dot(dynb, xk1).astype(_BF16)
        return jnp.concatenate([x01, ac[0:N], ac[N:2 * N]], axis=-1)

    wins = [xsb[L - 2], xsb[L - 1], z9]
    for s in range(STEPS):
        h1 = conv_one(wins[-3], wins[-2], wins[-1])
        f = (_dot(h1, Wg) + bg).astype(_BF16)
        fh_ref[0, s + 1] = _dot(h1, WgFk) + bfk2
        wins.append(f)
        for l in (2 * s, 2 * s + 1):
            if l < Lp - 1:
                backcast_ln(l, h_of_l(l))
    backcast_ln(Lp - 2, h_of_l(Lp - 2))


def kernel(X, X_spa, dynamic_graph, static_graph, W_fc, W_gcn, b_gcn, W_bc,
           b_bc, W_fk, b_fk, ln_scale, ln_bias):
    B, L, N, D = X_spa.shape
    Lp = L - K_T + 1
    S = SEQ_LENGTH // GAP

    WgT = W_gcn.T
    args = (
        X,
        X_spa.astype(_BF16),
        dynamic_graph,
        W_fc.T.astype(_BF16),
        WgT.astype(_BF16),
        (WgT @ W_bc.T).astype(_BF16),
        (WgT @ W_fk.T).astype(_BF16),
        b_gcn.reshape(1, D),
        (b_gcn @ W_bc.T + b_bc).reshape(1, D),
        (b_gcn @ W_fk.T + b_fk).reshape(1, FK_DIM),
        ln_scale.reshape(1, D),
        ln_bias.reshape(1, D),
    )

    full = lambda shape: pl.BlockSpec(shape, lambda b: (0,) * len(shape))
    batched = lambda shape: pl.BlockSpec(
        (1,) + shape[1:], lambda b: (b,) + (0,) * (len(shape) - 1))

    in_specs = [
        batched(X.shape),
        batched(X_spa.shape),
        pl.BlockSpec((K_S, 1, N, K_T * N), lambda b: (0, b, 0, 0)),
        full((K_T * D, K_T * D)),
        full((K_T * D, D)),
        full((K_T * D, D)),
        full((K_T * D, FK_DIM)),
        full((1, D)),
        full((1, D)),
        full((1, FK_DIM)),
        full((1, D)),
        full((1, D)),
    ]
    out_specs = [
        batched((B, Lp, N, D)),
        batched((B, S, N, FK_DIM)),
    ]
    out_shape = [
        jax.ShapeDtypeStruct((B, Lp, N, D), _F32),
        jax.ShapeDtypeStruct((B, S, N, FK_DIM), _F32),
    ]

    u, fh = pl.pallas_call(
        _dif_block_kernel,
        grid=(B,),
        in_specs=in_specs,
        out_specs=out_specs,
        out_shape=out_shape,
        scratch_shapes=[
            pltpu.VMEM((K_T * N, Lp * D), _BF16),
        ],
        compiler_params=pltpu.CompilerParams(
            dimension_semantics=("parallel",),
        ),
    )(*args)
    return (u, fh)


# speedup vs baseline: 1.1095x; 1.0663x over previous
import jax
import jax.numpy as jnp
from jax.experimental import pallas as pl
from jax.experimental.pallas import tpu as pltpu

K_T = 3
K_S = 2
HIDDEN = 64
FK_DIM = 256
SEQ_LENGTH = 12
GAP = 2

_F32 = jnp.float32


def _dot(a, b):
    return jnp.dot(a, b, preferred_element_type=_F32)


def _dif_block_kernel(x_ref, xspa_ref, dyn_ref, wfcT_ref, wgT_ref, bg_ref,
                      wbcT_ref, bbc_ref, wfkT_ref, bfk_ref, lns_ref, lnb_ref,
                      u_ref, fh_ref, xk_scr, x0_scr):
    L = xspa_ref.shape[1]
    N = xspa_ref.shape[2]
    D = HIDDEN
    Lp = L - K_T + 1
    STEPS = SEQ_LENGTH // GAP - 1

    WfcT = wfcT_ref[...]
    Wg = wgT_ref[...]
    bg = bg_ref[...]
    WbcT = wbcT_ref[...]
    bbc = bbc_ref[...]
    WfkT = wfkT_ref[...]
    bfk = bfk_ref[...]
    lns = lns_ref[...]
    lnb = lnb_ref[...]

    dynC = dyn_ref[...].reshape(K_S * N, K_T * N)

    for l in range(Lp):
        xr = jnp.concatenate(
            [xspa_ref[0, l + j] for j in range(K_T)], axis=-1)
        t = jnp.tanh(_dot(xr, WfcT))
        x0_scr[l] = (t[:, 0:D] + t[:, D:2 * D] + t[:, 2 * D:3 * D]) * (1.0 / K_T)
        for j in range(K_T):
            xk_scr[j * N:(j + 1) * N, l * D:(l + 1) * D] = t[:, j * D:(j + 1) * D]

    acat = _dot(dynC, xk_scr[...])

    zs_last = None
    for l in range(Lp):
        h = jnp.concatenate(
            [x0_scr[l], acat[0:N, l * D:(l + 1) * D],
             acat[N:2 * N, l * D:(l + 1) * D]], axis=-1)
        z = _dot(h, Wg) + bg
        if l == Lp - 1:
            zs_last = z
        bc = _dot(z, WbcT) + bbc
        v = x_ref[0, l + K_T - 1] - jnp.maximum(bc, 0.0)
        mu = jnp.mean(v, axis=-1, keepdims=True)
        var = jnp.mean((v - mu) * (v - mu), axis=-1, keepdims=True)
        u_ref[0, l] = (v - mu) * jax.lax.rsqrt(var + 1e-5) * lns + lnb

    def conv_one(w0, w1, w2):
        xr = jnp.concatenate([w0, w1, w2], axis=-1)
        t = jnp.tanh(_dot(xr, WfcT))
        x0 = (t[:, 0:D] + t[:, D:2 * D] + t[:, 2 * D:3 * D]) * (1.0 / K_T)
        xk = jnp.concatenate(
            [t[:, j * D:(j + 1) * D] for j in range(K_T)], axis=0)
        ac = _dot(dynC, xk)
        h = jnp.concatenate([x0, ac[0:N], ac[N:2 * N]], axis=-1)
        return _dot(h, Wg) + bg

    wins = [xspa_ref[0, L - 2], xspa_ref[0, L - 1], zs_last]
    frames = [zs_last]
    for _ in range(STEPS):
        f = conv_one(wins[-3], wins[-2], wins[-1])
        wins.append(f)
        frames.append(f)

    fcat = jnp.concatenate(frames, axis=0)
    fh = _dot(fcat, WfkT) + bfk
    fh_ref[0] = fh.reshape(STEPS + 1, N, FK_DIM)


def kernel(X, X_spa, dynamic_graph, static_graph, W_fc, W_gcn, b_gcn, W_bc,
           b_bc, W_fk, b_fk, ln_scale, ln_bias):
    B, L, N, D = X_spa.shape
    Lp = L - K_T + 1
    S = SEQ_LENGTH // GAP

    args = (
        X,
        X_spa,
        dynamic_graph,
        W_fc.T,
        W_gcn.T,
        b_gcn.reshape(1, D),
        W_bc.T,
        b_bc.reshape(1, D),
        W_fk.T,
        b_fk.reshape(1, FK_DIM),
        ln_scale.reshape(1, D),
        ln_bias.reshape(1, D),
    )

    full = lambda shape: pl.BlockSpec(shape, lambda b: (0,) * len(shape))
    batched = lambda shape: pl.BlockSpec(
        (1,) + shape[1:], lambda b: (b,) + (0,) * (len(shape) - 1))

    in_specs = [
        batched(X.shape),
        batched(X_spa.shape),
        pl.BlockSpec((K_S, 1, N, K_T * N), lambda b: (0, b, 0, 0)),
        full((K_T * D, K_T * D)),
        full((K_T * D, D)),
        full((1, D)),
        full((D, D)),
        full((1, D)),
        full((D, FK_DIM)),
        full((1, FK_DIM)),
        full((1, D)),
        full((1, D)),
    ]
    out_specs = [
        batched((B, Lp, N, D)),
        batched((B, S, N, FK_DIM)),
    ]
    out_shape = [
        jax.ShapeDtypeStruct((B, Lp, N, D), _F32),
        jax.ShapeDtypeStruct((B, S, N, FK_DIM), _F32),
    ]

    u, fh = pl.pallas_call(
        _dif_block_kernel,
        grid=(B,),
        in_specs=in_specs,
        out_specs=out_specs,
        out_shape=out_shape,
        scratch_shapes=[
            pltpu.VMEM((K_T * N, Lp * D), _F32),
            pltpu.VMEM((Lp, N, D), _F32),
        ],
        compiler_params=pltpu.CompilerParams(
            dimension_semantics=("parallel",),
        ),
    )(*args)
    return (u, fh)
